# Initial kernel scaffold; baseline (speedup 1.0000x reference)
import jax
import jax.numpy as jnp
from jax.experimental import pallas as pl


def kernel(input_ids, position_ids, type_ids, word_emb, pos_emb, type_emb, W, b, gamma, beta):
    raise NotImplementedError("write your pallas kernel here")


# speedup vs baseline: 2.1861x; 2.1861x over previous
import functools

import jax
import jax.numpy as jnp
from jax import lax
from jax.experimental import pallas as pl
from jax.experimental.pallas import tpu as pltpu
from jax.experimental.pallas import tpu_sc as plsc

_VOCAB = 100000
_EMB = 1024
_DIM = 4096
_B, _S = 4, 4096
_NTOK = _B * _S

_NW = 32
_CW = 32
_NCHUNK = _NTOK // (_NW * _CW)

_TM = 256


def _sc_gather_two(idx_w3, idx_p3, word_emb, pos_emb):
    mesh = plsc.VectorSubcoreMesh(core_axis_name="c", subcore_axis_name="s")
    nc = 2

    @functools.partial(
        pl.kernel,
        mesh=mesh,
        out_type=(
            jax.ShapeDtypeStruct((_NTOK, _EMB), jnp.float32),
            jax.ShapeDtypeStruct((_NTOK, _EMB), jnp.float32),
        ),
        scratch_types=[
            pltpu.VMEM((_NCHUNK, _CW), jnp.int32),
            pltpu.VMEM((_NCHUNK, _CW), jnp.int32),
            pltpu.VMEM((_CW, _EMB), jnp.float32),
            pltpu.VMEM((_CW, _EMB), jnp.float32),
            pltpu.SemaphoreType.DMA,
            pltpu.SemaphoreType.DMA,
        ],
    )
    def k(idxw_hbm, idxp_hbm, wtab_hbm, ptab_hbm, xw_hbm, xp_hbm,
          idxw_v, idxp_v, bufw, bufp, semw, semp):
        wid = lax.axis_index("s") * nc + lax.axis_index("c")
        pltpu.sync_copy(idxw_hbm.at[wid], idxw_v)
        pltpu.sync_copy(idxp_hbm.at[wid], idxp_v)
        tok0 = wid * (_NCHUNK * _CW)
        for c in range(_NCHUNK):
            cw = pltpu.async_copy(wtab_hbm.at[idxw_v.at[c]], bufw, semw)
            cp = pltpu.async_copy(ptab_hbm.at[idxp_v.at[c]], bufp, semp)
            base = tok0 + c * _CW
            cw.wait()
            pltpu.sync_copy(bufw, xw_hbm.at[pl.ds(base, _CW)])
            cp.wait()
            pltpu.sync_copy(bufp, xp_hbm.at[pl.ds(base, _CW)])

    return k(idx_w3, idx_p3, word_emb, pos_emb)


def _tc_body(xw_ref, xp_ref, t_ref, te_ref, w_ref, b_ref, g_ref, be_ref, o_ref):
    x = xw_ref[...] + xp_ref[...]
    te = te_ref[...]
    t = t_ref[...].astype(jnp.float32)
    x = x + te[0:1, :] + t * (te[1:2, :] - te[0:1, :])
    y = jnp.dot(x, w_ref[...], preferred_element_type=jnp.float32)
    y = y + b_ref[...]
    mu = jnp.mean(y, axis=1, keepdims=True)
    yc = y - mu
    var = jnp.mean(yc * yc, axis=1, keepdims=True)
    o_ref[...] = yc * lax.rsqrt(var + 1e-5) * g_ref[...] + be_ref[...]


def _tc_project_ln(xw, xp, type_ids_col, type_emb, W, b2, g2, be2):
    grid = (_NTOK // _TM,)
    return pl.pallas_call(
        _tc_body,
        grid=grid,
        in_specs=[
            pl.BlockSpec((_TM, _EMB), lambda i: (i, 0)),
            pl.BlockSpec((_TM, _EMB), lambda i: (i, 0)),
            pl.BlockSpec((_TM, 1), lambda i: (i, 0)),
            pl.BlockSpec((2, _EMB), lambda i: (0, 0)),
            pl.BlockSpec((_EMB, _DIM), lambda i: (0, 0)),
            pl.BlockSpec((1, _DIM), lambda i: (0, 0)),
            pl.BlockSpec((1, _DIM), lambda i: (0, 0)),
            pl.BlockSpec((1, _DIM), lambda i: (0, 0)),
        ],
        out_specs=pl.BlockSpec((_TM, _DIM), lambda i: (i, 0)),
        out_shape=jax.ShapeDtypeStruct((_NTOK, _DIM), jnp.float32),
    )(xw, xp, type_ids_col, type_emb, W, b2, g2, be2)


def kernel(input_ids, position_ids, type_ids, word_emb, pos_emb, type_emb,
           W, b, gamma, beta):
    idx_w3 = input_ids.reshape(_NW, _NCHUNK, _CW)
    idx_p3 = position_ids.reshape(_NW, _NCHUNK, _CW)
    xw, xp = _sc_gather_two(idx_w3, idx_p3, word_emb, pos_emb)
    out = _tc_project_ln(
        xw, xp,
        type_ids.reshape(_NTOK, 1),
        type_emb,
        W,
        b.reshape(1, _DIM),
        gamma.reshape(1, _DIM),
        beta.reshape(1, _DIM),
    )
    return out.reshape(_B, _S, _DIM)


# speedup vs baseline: 2.2446x; 1.0268x over previous
import functools

import jax
import jax.numpy as jnp
from jax import lax
from jax.experimental import pallas as pl
from jax.experimental.pallas import tpu as pltpu
from jax.experimental.pallas import tpu_sc as plsc

_VOCAB = 100000
_EMB = 1024
_DIM = 4096
_B, _S = 4, 4096
_NTOK = _B * _S

_NW = 32
_CW = 32
_NCHUNK = _NTOK // (_NW * _CW)

_TM = 256


def _sc_gather_two(idx_w3, idx_p3, word_emb, pos_emb):
    mesh = plsc.VectorSubcoreMesh(core_axis_name="c", subcore_axis_name="s")
    nc = 2

    @functools.partial(
        pl.kernel,
        mesh=mesh,
        out_type=(
            jax.ShapeDtypeStruct((_NTOK, _EMB), jnp.float32),
            jax.ShapeDtypeStruct((_NTOK, _EMB), jnp.float32),
        ),
        scratch_types=[
            pltpu.VMEM((_NCHUNK, _CW), jnp.int32),
            pltpu.VMEM((_NCHUNK, _CW), jnp.int32),
            pltpu.VMEM((_CW, _EMB), jnp.float32),
            pltpu.VMEM((_CW, _EMB), jnp.float32),
            pltpu.SemaphoreType.DMA,
            pltpu.SemaphoreType.DMA,
        ],
    )
    def k(idxw_hbm, idxp_hbm, wtab_hbm, ptab_hbm, xw_hbm, xp_hbm,
          idxw_v, idxp_v, bufw, bufp, semw, semp):
        wid = lax.axis_index("s") * nc + lax.axis_index("c")
        pltpu.sync_copy(idxw_hbm.at[wid], idxw_v)
        pltpu.sync_copy(idxp_hbm.at[wid], idxp_v)
        tok0 = wid * (_NCHUNK * _CW)
        for c in range(_NCHUNK):
            cw = pltpu.async_copy(wtab_hbm.at[idxw_v.at[c]], bufw, semw)
            cp = pltpu.async_copy(ptab_hbm.at[idxp_v.at[c]], bufp, semp)
            base = tok0 + c * _CW
            cw.wait()
            pltpu.sync_copy(bufw, xw_hbm.at[pl.ds(base, _CW)])
            cp.wait()
            pltpu.sync_copy(bufp, xp_hbm.at[pl.ds(base, _CW)])

    return k(idx_w3, idx_p3, word_emb, pos_emb)


def _tc_body(xw_ref, xp_ref, t_ref, te_ref, w_ref, b_ref, g_ref, be_ref, o_ref):
    x = xw_ref[...] + xp_ref[...]
    te = te_ref[...]
    t = t_ref[...].astype(jnp.float32)
    x = x + te[0:1, :] + t * (te[1:2, :] - te[0:1, :])
    y = jnp.dot(x.astype(jnp.bfloat16), w_ref[...],
                preferred_element_type=jnp.float32)
    y = y + b_ref[...]
    mu = jnp.mean(y, axis=1, keepdims=True)
    m2 = jnp.mean(y * y, axis=1, keepdims=True)
    r = lax.rsqrt(m2 - mu * mu + 1e-5)
    scale = r * g_ref[...]
    o_ref[...] = y * scale + (be_ref[...] - mu * scale)


def _tc_project_ln(xw, xp, type_ids_col, type_emb, W, b2, g2, be2):
    grid = (_NTOK // _TM,)
    return pl.pallas_call(
        _tc_body,
        grid=grid,
        in_specs=[
            pl.BlockSpec((_TM, _EMB), lambda i: (i, 0)),
            pl.BlockSpec((_TM, _EMB), lambda i: (i, 0)),
            pl.BlockSpec((_TM, 1), lambda i: (i, 0)),
            pl.BlockSpec((2, _EMB), lambda i: (0, 0)),
            pl.BlockSpec((_EMB, _DIM), lambda i: (0, 0)),
            pl.BlockSpec((1, _DIM), lambda i: (0, 0)),
            pl.BlockSpec((1, _DIM), lambda i: (0, 0)),
            pl.BlockSpec((1, _DIM), lambda i: (0, 0)),
        ],
        out_specs=pl.BlockSpec((_TM, _DIM), lambda i: (i, 0)),
        out_shape=jax.ShapeDtypeStruct((_NTOK, _DIM), jnp.float32),
    )(xw, xp, type_ids_col, type_emb, W, b2, g2, be2)


def kernel(input_ids, position_ids, type_ids, word_emb, pos_emb, type_emb,
           W, b, gamma, beta):
    idx_w3 = input_ids.reshape(_NW, _NCHUNK, _CW)
    idx_p3 = position_ids.reshape(_NW, _NCHUNK, _CW)
    xw, xp = _sc_gather_two(idx_w3, idx_p3, word_emb, pos_emb)
    out = _tc_project_ln(
        xw, xp,
        type_ids.reshape(_NTOK, 1),
        type_emb,
        W.astype(jnp.bfloat16),
        b.reshape(1, _DIM),
        gamma.reshape(1, _DIM),
        beta.reshape(1, _DIM),
    )
    return out.reshape(_B, _S, _DIM)
